# baseline (speedup 1.0000x reference)
import functools

import jax
import jax.numpy as jnp
from jax import lax
from jax.experimental import pallas as pl
from jax.experimental.pallas import tpu as pltpu
from jax.experimental.pallas import tpu_sc as plsc


def _z_body(x_ref, w_ref, g_ref, bias_ref, z_ref):
    EB = x_ref.shape[1]
    C = x_ref.shape[2]
    bias = bias_ref[...]
    for e in range(EB):
        xe = x_ref[0, e]
        we = w_ref[e]
        ze = jnp.dot(xe, we, preferred_element_type=jnp.float32)
        gg = g_ref[0, 0, e * C:(e + 1) * C]
        z_ref[0, e * C:(e + 1) * C, :] = (ze + bias) * gg[:, None]


def _compute_z(x, Wt, g, bias2, B, E, C, I, O, R):
    EB = 16
    return pl.pallas_call(
        _z_body,
        grid=(B, E // EB),
        in_specs=[
            pl.BlockSpec((1, EB, C, I), lambda b, ec: (b, ec, 0, 0)),
            pl.BlockSpec((EB, I, O), lambda b, ec: (ec, 0, 0)),
            pl.BlockSpec((1, 1, EB * C), lambda b, ec: (b, 0, ec)),
            pl.BlockSpec((1, O), lambda b, ec: (0, 0)),
        ],
        out_specs=pl.BlockSpec((1, EB * C, O), lambda b, ec: (b, ec, 0)),
        out_shape=jax.ShapeDtypeStruct((B, R, O), jnp.float32),
        compiler_params=pltpu.CompilerParams(
            dimension_semantics=("parallel", "arbitrary"),
        ),
    )(x, Wt, g, bias2)


def _scatter_sc(z2, idx2, B, T, O, R):
    NS = 8
    OS = O // NS
    TPW = T // 16
    K = 64
    BPC = B // 2
    LCAP = R + 16
    mesh = plsc.VectorSubcoreMesh(
        core_axis_name="c", subcore_axis_name="s", num_cores=2,
        num_subcores=16)

    @functools.partial(
        pl.kernel,
        out_type=jax.ShapeDtypeStruct((B * T, O), jnp.float32),
        mesh=mesh,
        scratch_types=[
            pltpu.VMEM((R,), jnp.int32),
            pltpu.VMEM((LCAP,), jnp.int32),
            pltpu.VMEM((LCAP,), jnp.int32),
            pltpu.VMEM((K, OS), jnp.float32),
            pltpu.VMEM((TPW, OS), jnp.float32),
        ],
        compiler_params=pltpu.CompilerParams(needs_layout_passes=False),
    )
    def k(z_hbm, idx_hbm, out_hbm, idxall_v, match_v, tloc_v, buf_v, acc_v):
        cid = lax.axis_index("c")
        sid = lax.axis_index("s")
        lo = sid * TPW
        zero16 = jnp.zeros((16,), jnp.float32)
        for bl in range(BPC):
            b = cid * BPC + bl
            pltpu.sync_copy(idx_hbm.at[b], idxall_v)

            def prefill(i, _):
                match_v[pl.ds(i * 16, 16)] = jnp.full((16,), b * R, jnp.int32)
                tloc_v[pl.ds(i * 16, 16)] = jnp.zeros((16,), jnp.int32)
                return 0
            lax.fori_loop(0, LCAP // 16, prefill, 0, unroll=4)

            def scan(i, off):
                v = idxall_v[pl.ds(i * 16, 16)]
                m = (v >= lo) & (v < lo + TPW)
                mi = jnp.where(m, 1, 0).astype(jnp.int32)
                pos = off + plsc.cumsum(mi) - mi
                rid = lax.iota(jnp.int32, 16) + (i * 16 + b * R)
                plsc.store_scatter(match_v, [pos], rid, mask=m)
                plsc.store_scatter(tloc_v, [pos], v - lo, mask=m)
                return off + plsc.all_reduce_population_count(m)[0]
            n = lax.fori_loop(0, R // 16, scan, 0)
            nch = (n + K - 1) // K

            for ns in range(NS):
                def zero_all(i, _):
                    r0 = i // (OS // 16)
                    c0 = (i % (OS // 16)) * 16
                    acc_v[r0, pl.ds(c0, 16)] = zero16
                    return 0
                lax.fori_loop(0, TPW * OS // 16, zero_all, 0, unroll=8)

                def chunk(g, _):
                    pltpu.sync_copy(
                        z_hbm.at[match_v.at[pl.ds(g * K, K)],
                                 pl.ds(ns * OS, OS)],
                        buf_v)
                    mg = jnp.minimum(K, n - g * K)

                    def row(i2, _2):
                        tv = tloc_v[pl.ds(g * K + i2, 16)]
                        tok = lax.min(lax.max(tv[0], 0), TPW - 1)
                        for cc in range(OS // 16):
                            sl = pl.ds(cc * 16, 16)
                            acc_v[tok, sl] = acc_v[tok, sl] + buf_v[i2, sl]
                        return 0
                    lax.fori_loop(0, mg, row, 0)
                    return 0
                lax.fori_loop(0, nch, chunk, 0)
                pltpu.sync_copy(
                    acc_v,
                    out_hbm.at[pl.ds(b * T + lo, TPW), pl.ds(ns * OS, OS)])

    return k(z2, idx2)


def kernel(x_expert, expert_indices, expert_gate, num_tokens, W, b):
    B, E, C, I = x_expert.shape
    O = W.shape[0]
    R = E * C
    T = num_tokens if isinstance(num_tokens, int) else R

    Wt = W.reshape(E, O, I).transpose(0, 2, 1)
    g_f = expert_gate.reshape(B, 1, R)
    bias2 = b.reshape(1, O)
    z = _compute_z(x_expert, Wt, g_f, bias2, B, E, C, I, O, R)

    z2 = z.reshape(B * R, O)
    idx2 = expert_indices.reshape(B, R)
    out = _scatter_sc(z2, idx2, B, T, O, R)
    return out.reshape(B, T, O)
